# baseline (speedup 1.0000x reference)
import functools

import jax
import jax.numpy as jnp
from jax import lax
from jax.experimental import pallas as pl
from jax.experimental.pallas import tpu as pltpu
from jax.experimental.pallas import tpu_sc as plsc

_EMBED_DIM = 64
_CHUNK = 1024


def _gather_body(n_workers, n_per_w, idx_hbm, table_hbm, out_hbm,
                 idx_v, rows_v, sem):
    nc = 2
    wid = lax.axis_index("s") * nc + lax.axis_index("c")
    base = wid * n_per_w
    n_chunks = n_per_w // _CHUNK

    def step(g, carry):
        off = base + g * _CHUNK
        pltpu.sync_copy(idx_hbm.at[pl.ds(off, _CHUNK)], idx_v)
        pltpu.async_copy(table_hbm.at[idx_v], rows_v, sem).wait()
        pltpu.sync_copy(rows_v, out_hbm.at[pl.ds(off, _CHUNK)])
        return carry

    lax.fori_loop(0, n_chunks, step, 0)


def kernel(x, table):
    b, s = x.shape
    flat = x.reshape(-1).astype(jnp.int32)
    n = flat.shape[0]

    info = plsc.get_sparse_core_info()
    n_workers = info.num_cores * info.num_subcores
    assert n % (n_workers * _CHUNK) == 0
    n_per_w = n // n_workers

    mesh = plsc.VectorSubcoreMesh(core_axis_name="c", subcore_axis_name="s")
    run = functools.partial(
        pl.kernel,
        mesh=mesh,
        out_type=jax.ShapeDtypeStruct((n, _EMBED_DIM), table.dtype),
        scratch_types=[
            pltpu.VMEM((_CHUNK,), jnp.int32),
            pltpu.VMEM((_CHUNK, _EMBED_DIM), jnp.float32),
            pltpu.SemaphoreType.DMA,
        ],
        compiler_params=pltpu.CompilerParams(use_tc_tiling_on_sc=False),
    )(functools.partial(_gather_body, n_workers, n_per_w))
    out = run(flat, table)
    return out.reshape(b, s, _EMBED_DIM)


# speedup vs baseline: 1.0180x; 1.0180x over previous
import functools

import jax
import jax.numpy as jnp
from jax import lax
from jax.experimental import pallas as pl
from jax.experimental.pallas import tpu as pltpu
from jax.experimental.pallas import tpu_sc as plsc

_EMBED_DIM = 64
_CHUNK = 320
_NBUF = 4


def _gather_body(n_per_w, idx_hbm, table_hbm, out_hbm,
                 idx_all, rows, gsems, osems):
    nc = 2
    wid = lax.axis_index("s") * nc + lax.axis_index("c")
    base = wid * n_per_w
    n_chunks = n_per_w // _CHUNK

    pltpu.sync_copy(idx_hbm.at[pl.ds(base, n_per_w)], idx_all)

    def idx_at(k):
        return idx_all.at[pl.ds(k * _CHUNK, _CHUNK)]

    def out_at(k):
        return out_hbm.at[pl.ds(base + k * _CHUNK, _CHUNK)]

    def gather(k, b):
        return pltpu.make_async_copy(table_hbm.at[idx_at(k)], rows[b],
                                     gsems[b])

    def scatter(k, b):
        return pltpu.make_async_copy(rows[b], out_at(k), osems[b])

    for b in range(_NBUF - 1):
        gather(b, b).start()

    def step(o, carry):
        for b in range(_NBUF):
            k = o * _NBUF + b
            bp = (b + _NBUF - 1) % _NBUF
            gather(k, b).wait()
            scatter(k, b).start()
            j = k + _NBUF - 1

            @pl.when(j < n_chunks)
            def _():
                @pl.when(k >= 1)
                def _():
                    scatter(k - 1, bp).wait()
                gather(j, bp).start()

        return carry

    lax.fori_loop(0, n_chunks // _NBUF, step, 0)

    for b in range(_NBUF):
        scatter(n_chunks - _NBUF + b, b).wait()


def kernel(x, table):
    b, s = x.shape
    flat = x.reshape(-1).astype(jnp.int32)
    n = flat.shape[0]

    info = plsc.get_sparse_core_info()
    n_workers = info.num_cores * info.num_subcores
    n_per_w = n // n_workers
    assert n % n_workers == 0
    assert n_per_w % (_CHUNK * _NBUF) == 0

    mesh = plsc.VectorSubcoreMesh(core_axis_name="c", subcore_axis_name="s")
    run = functools.partial(
        pl.kernel,
        mesh=mesh,
        out_type=jax.ShapeDtypeStruct((n, _EMBED_DIM), table.dtype),
        scratch_types=[
            pltpu.VMEM((n_per_w,), jnp.int32),
            [pltpu.VMEM((_CHUNK, _EMBED_DIM), jnp.float32)
             for _ in range(_NBUF)],
            [pltpu.SemaphoreType.DMA for _ in range(_NBUF)],
            [pltpu.SemaphoreType.DMA for _ in range(_NBUF)],
        ],
        compiler_params=pltpu.CompilerParams(use_tc_tiling_on_sc=False),
    )(functools.partial(_gather_body, n_per_w))
    out = run(flat, table)
    return out.reshape(b, s, _EMBED_DIM)
